# baseline (speedup 1.0000x reference)
import functools

import jax
import jax.numpy as jnp
from jax.experimental import pallas as pl

_N = 10000
_D = 128
_K = 4
_POOL = 0.5
_MM_BLK = 400


def _with_self_loops(ei, num_nodes):
    loops = jnp.arange(num_nodes, dtype=ei.dtype)
    return jnp.concatenate([ei, jnp.stack([loops, loops])], axis=1)


def _mm_kernel(x_ref, w_ref, a2_ref, h0, h1, h2, h3, sd_ref):
    xb = x_ref[...]
    sd = jnp.zeros((_MM_BLK, _D), jnp.float32)
    for k, href in enumerate((h0, h1, h2, h3)):
        hk = jnp.dot(xb, w_ref[k], preferred_element_type=jnp.float32)
        href[...] = hk
        sd = sd + jnp.dot(hk, a2_ref[k], preferred_element_type=jnp.float32)
    sd_ref[...] = sd


@jax.jit
def _dense_stage(x, W, A2):
    grid = (_N // _MM_BLK,)
    outs = pl.pallas_call(
        _mm_kernel,
        out_shape=[jax.ShapeDtypeStruct((_N, _D), jnp.float32)] * 5,
        grid=grid,
        in_specs=[
            pl.BlockSpec((_MM_BLK, _D), lambda i: (i, 0)),
            pl.BlockSpec((_K, _D, _D), lambda i: (0, 0, 0)),
            pl.BlockSpec((_K, _D, _D), lambda i: (0, 0, 0)),
        ],
        out_specs=[pl.BlockSpec((_MM_BLK, _D), lambda i: (i, 0))] * 5,
    )(x, W, A2)
    return outs[:4], outs[4]


def _ew_call(fn, n_out, *arrs):
    e = arrs[0].shape[0]
    rows = -(-e // (256 * _D)) * 256
    ep = rows * _D
    grid = (rows // 256,)
    padded = [jnp.pad(a, (0, ep - e), constant_values=1.0).reshape(rows, _D)
              for a in arrs]
    outs = pl.pallas_call(
        fn,
        out_shape=[jax.ShapeDtypeStruct((rows, _D), jnp.float32)] * n_out,
        grid=grid,
        in_specs=[pl.BlockSpec((256, _D), lambda i: (i, 0))] * len(arrs),
        out_specs=[pl.BlockSpec((256, _D), lambda i: (i, 0))] * n_out,
    )(*padded)
    if n_out == 1:
        return outs[0].reshape(-1)[:e]
    return [o.reshape(-1)[:e] for o in outs]


def _logit_kernel(*refs):
    ins, outs = refs[:8], refs[8:]
    for k in range(_K):
        z = ins[k][...] + ins[4 + k][...]
        outs[k][...] = jnp.where(z >= 0, z, jnp.float32(0.2) * z)


def _exp_kernel(*refs):
    ins, outs = refs[:8], refs[8:]
    for k in range(_K):
        outs[k][...] = jnp.exp(ins[k][...] - ins[4 + k][...])


def _attn_kernel(*refs):
    ins, outs = refs[:8], refs[8:]
    att = []
    for k in range(_K):
        a = ins[k][...] / (ins[4 + k][...] + jnp.float32(1e-16))
        att.append(a)
        outs[k][...] = a
    outs[4][...] = (((att[0] + att[1]) + att[2]) + att[3]) / jnp.float32(4.0)


def _meag(x, ei, W, As, Ad):
    src, dst = ei[0], ei[1]
    A2 = jnp.zeros((_K, _D, _D), jnp.float32)
    for k in range(_K):
        A2 = A2.at[k, :, 2 * k].set(As[k]).at[k, :, 2 * k + 1].set(Ad[k])
    hs, sd = _dense_stage(x, W, A2)

    sa = [sd[:, 2 * k][src] for k in range(_K)]
    da = [sd[:, 2 * k + 1][dst] for k in range(_K)]
    e = _ew_call(_logit_kernel, _K, *sa, *da)

    emaxg = []
    for k in range(_K):
        emax = jax.ops.segment_max(e[k], dst, num_segments=_N)
        emax = jnp.where(jnp.isfinite(emax), emax, 0.0)
        emaxg.append(emax[dst])
    ex = _ew_call(_exp_kernel, _K, *e, *emaxg)

    dg = []
    for k in range(_K):
        denom = jax.ops.segment_sum(ex[k], dst, num_segments=_N)
        dg.append(denom[dst])
    res = _ew_call(_attn_kernel, _K + 1, *ex, *dg)
    attn, attn_mean = res[:_K], res[_K]

    outs = []
    for k in range(_K):
        outs.append(jax.ops.segment_sum(attn[k][:, None] * hs[k][src], dst,
                                        num_segments=_N))
    out = (((outs[0] + outs[1]) + outs[2]) + outs[3]) / 4.0
    return out, attn_mean


def kernel(x, edge_index, W1, W2, W3, As1, Ad1, As2, Ad2, As3, Ad3):
    params = [(W1, As1, Ad1), (W2, As2, Ad2), (W3, As3, Ad3)]
    edge_list = []
    ei = _with_self_loops(edge_index, x.shape[0])
    for i in range(3):
        edge_list.append(ei)
        x, attn = _meag(x, ei, *params[i])
        x = jax.nn.leaky_relu(x, 0.01)
        x = x / jnp.maximum(jnp.linalg.norm(x, axis=0, keepdims=True), 1e-12)
        kk = max(int(attn.shape[0] * _POOL), 1)
        _, idx = jax.lax.top_k(attn, kk)
        ei = _with_self_loops(ei[:, idx], x.shape[0])
    return (x, ei) + tuple(edge_list)
